# baseline (speedup 1.0000x reference)
import functools
import jax
import jax.numpy as jnp
from jax import lax
from jax.experimental import pallas as pl
from jax.experimental.pallas import tpu as pltpu
from jax.experimental.pallas import tpu_sc as plsc

_NC, _NS = 2, 16
_C = 16
_S_BLK = 512


def _sc_last_batch(x2d, table, B, S, D):
    NW = _NC * _NS
    s_per_w = S // NW
    chunks = s_per_w // _C
    row_off = (B - 1) * S

    mesh = plsc.VectorSubcoreMesh(core_axis_name="c", subcore_axis_name="s")

    @functools.partial(
        pl.kernel,
        mesh=mesh,
        out_type=jax.ShapeDtypeStruct((S, D), jnp.float32),
        scratch_types=[
            pltpu.VMEM((_C, D), jnp.float32),
            pltpu.VMEM((_C, D), jnp.float32),
            pltpu.VMEM((_C, D), jnp.float32),
            pltpu.VMEM((_C, D), jnp.float32),
            pltpu.SemaphoreType.DMA,
            pltpu.SemaphoreType.DMA,
            pltpu.SemaphoreType.DMA,
            pltpu.SemaphoreType.DMA,
        ],
        compiler_params=pltpu.CompilerParams(use_tc_tiling_on_sc=True),
    )
    def sc_add(x_hbm, t_hbm, out_hbm, xb0, xb1, tb0, tb1, is0, is1, os0, os1):
        wid = lax.axis_index("s") * _NC + lax.axis_index("c")
        s0 = wid * s_per_w
        xbufs, tbufs = (xb0, xb1), (tb0, tb1)
        isems, osems = (is0, is1), (os0, os1)

        def start_in(j, b):
            sbase = s0 + j * _C
            pltpu.async_copy(t_hbm.at[pl.ds(sbase, _C)], tbufs[b], isems[b])
            pltpu.async_copy(
                x_hbm.at[pl.ds(row_off + sbase, _C)], xbufs[b], isems[b])

        def wait_in(j, b):
            sbase = s0 + j * _C
            pltpu.make_async_copy(
                t_hbm.at[pl.ds(sbase, _C)], tbufs[b], isems[b]).wait()
            pltpu.make_async_copy(
                x_hbm.at[pl.ds(row_off + sbase, _C)], xbufs[b],
                isems[b]).wait()

        def start_out(j, b):
            sbase = s0 + j * _C
            pltpu.async_copy(xbufs[b], out_hbm.at[pl.ds(sbase, _C)], osems[b])

        def wait_out(j, b):
            sbase = s0 + j * _C
            pltpu.make_async_copy(
                xbufs[b], out_hbm.at[pl.ds(sbase, _C)], osems[b]).wait()

        start_in(0, 0)

        def half_step(jj, b):
            j = jj * 2 + b
            xb, tb = xbufs[b], tbufs[b]

            @pl.when(j + 1 < chunks)
            def _():
                @pl.when(j >= 1)
                def _():
                    wait_out(j - 1, 1 - b)
                start_in(j + 1, 1 - b)

            wait_in(j, b)

            @plsc.parallel_loop(0, D, 16, unroll=2)
            def _(i):
                sl = pl.ds(i, 16)
                for r in range(_C):
                    xb[r, sl] = xb[r, sl] + tb[r, sl]

            start_out(j, b)

        def body(jj, carry):
            half_step(jj, 0)
            half_step(jj, 1)
            return carry

        lax.fori_loop(0, chunks // 2, body, 0)
        wait_out(chunks - 2, 0)
        wait_out(chunks - 1, 1)

    return sc_add(x2d, table)


def _tc_add_kernel(x_ref, t_ref, o_ref):
    o_ref[...] = x_ref[...] + t_ref[...][None, :, :]


def _tc_first_batches(x, table, B, S, D):
    grid = (B - 1, S // _S_BLK)
    return pl.pallas_call(
        _tc_add_kernel,
        grid=grid,
        in_specs=[
            pl.BlockSpec((1, _S_BLK, D), lambda b, i: (b, i, 0)),
            pl.BlockSpec((_S_BLK, D), lambda b, i: (i, 0)),
        ],
        out_specs=pl.BlockSpec((1, _S_BLK, D), lambda b, i: (b, i, 0)),
        out_shape=jax.ShapeDtypeStruct((B - 1, S, D), jnp.float32),
        compiler_params=pltpu.CompilerParams(
            dimension_semantics=("parallel", "parallel"),
        ),
    )(x, table)


def kernel(input_embeddings, table):
    B, S, D = input_embeddings.shape
    x2d = input_embeddings.reshape(B * S, D)
    sc_out = _sc_last_batch(x2d, table, B, S, D)
    tc_out = _tc_first_batches(input_embeddings, table, B, S, D)
    return jnp.concatenate([tc_out, sc_out[None]], axis=0)


# speedup vs baseline: 1.3935x; 1.3935x over previous
import functools
import jax
import jax.numpy as jnp
from jax import lax
from jax.experimental import pallas as pl
from jax.experimental.pallas import tpu as pltpu
from jax.experimental.pallas import tpu_sc as plsc

_NC, _NS = 2, 16
_C = 16
_PB = 2


def kernel(input_embeddings, table):
    B, S, D = input_embeddings.shape
    BS = B * S
    NW = _NC * _NS
    s_per_w = S // NW
    chunks = s_per_w // _C
    nsub = B // _PB
    steps = chunks * nsub
    PC = _PB * _C

    mesh = plsc.VectorSubcoreMesh(core_axis_name="c", subcore_axis_name="s")

    @functools.partial(
        pl.kernel,
        mesh=mesh,
        out_type=jax.ShapeDtypeStruct((BS, D), jnp.float32),
        scratch_types=[
            pltpu.VMEM((PC, D), jnp.float32),
            pltpu.VMEM((PC, D), jnp.float32),
            pltpu.VMEM((_C, D), jnp.float32),
            pltpu.VMEM((_C, D), jnp.float32),
            pltpu.SemaphoreType.DMA,
            pltpu.SemaphoreType.DMA,
            pltpu.SemaphoreType.DMA,
            pltpu.SemaphoreType.DMA,
        ],
        compiler_params=pltpu.CompilerParams(use_tc_tiling_on_sc=True),
    )
    def sc_add(x_hbm, t_hbm, out_hbm, xb0, xb1, tb0, tb1, is0, is1, os0, os1):
        wid = lax.axis_index("s") * _NC + lax.axis_index("c")
        s0 = wid * s_per_w
        xbufs, tbufs = (xb0, xb1), (tb0, tb1)
        isems, osems = (is0, is1), (os0, os1)

        def start_in(j, b, sub, tpar):
            sbase = s0 + (j // nsub) * _C
            b0 = sub * _PB

            if sub == 0:
                pltpu.async_copy(
                    t_hbm.at[pl.ds(sbase, _C)], tbufs[tpar], isems[b])

            for p in range(_PB):
                pltpu.async_copy(
                    x_hbm.at[pl.ds((b0 + p) * S + sbase, _C)],
                    xbufs[b].at[pl.ds(p * _C, _C)], isems[b])

        def wait_in(j, b, sub, tpar):
            sbase = s0 + (j // nsub) * _C
            b0 = sub * _PB

            if sub == 0:
                pltpu.make_async_copy(
                    t_hbm.at[pl.ds(sbase, _C)], tbufs[tpar], isems[b]).wait()

            for p in range(_PB):
                pltpu.make_async_copy(
                    x_hbm.at[pl.ds((b0 + p) * S + sbase, _C)],
                    xbufs[b].at[pl.ds(p * _C, _C)], isems[b]).wait()

        def start_out(j, b, sub):
            sbase = s0 + (j // nsub) * _C
            b0 = sub * _PB
            for p in range(_PB):
                pltpu.async_copy(
                    xbufs[b].at[pl.ds(p * _C, _C)],
                    out_hbm.at[pl.ds((b0 + p) * S + sbase, _C)], osems[b])

        def wait_out(j, b, sub):
            sbase = s0 + (j // nsub) * _C
            b0 = sub * _PB
            for p in range(_PB):
                pltpu.make_async_copy(
                    xbufs[b].at[pl.ds(p * _C, _C)],
                    out_hbm.at[pl.ds((b0 + p) * S + sbase, _C)],
                    osems[b]).wait()

        start_in(0, 0, 0, 0)

        def sub_step(jj, off):
            j = jj * 4 + off
            b = off % 2
            sub = off % nsub
            tpar = (off // 2) % 2
            xb, tb = xbufs[b], tbufs[tpar]

            @pl.when(j + 1 < steps)
            def _():
                @pl.when(j >= 1)
                def _():
                    wait_out(j - 1, 1 - b, (off + 1) % nsub)
                start_in(j + 1, 1 - b, (off + 1) % nsub, ((off + 1) // 2) % 2)

            wait_in(j, b, sub, tpar)

            @plsc.parallel_loop(0, D, 16, unroll=2)
            def _(i):
                sl = pl.ds(i, 16)
                for p in range(_PB):
                    for r in range(_C):
                        xb[p * _C + r, sl] = xb[p * _C + r, sl] + tb[r, sl]

            start_out(j, b, sub)

        def body(jj, carry):
            for off in range(4):
                sub_step(jj, off)
            return carry

        lax.fori_loop(0, steps // 4, body, 0)
        wait_out(steps - 2, 0, 0)
        wait_out(steps - 1, 1, 1)

    out = sc_add(input_embeddings.reshape(BS, D), table)
    return out.reshape(B, S, D)


# speedup vs baseline: 1.4709x; 1.0555x over previous
import functools
import jax
import jax.numpy as jnp
from jax import lax
from jax.experimental import pallas as pl
from jax.experimental.pallas import tpu as pltpu
from jax.experimental.pallas import tpu_sc as plsc

_NC, _NS = 2, 16
_C = 8


def kernel(input_embeddings, table):
    B, S, D = input_embeddings.shape
    BS = B * S
    NW = _NC * _NS
    s_per_w = S // NW
    chunks = s_per_w // _C
    BC = B * _C

    mesh = plsc.VectorSubcoreMesh(core_axis_name="c", subcore_axis_name="s")

    @functools.partial(
        pl.kernel,
        mesh=mesh,
        out_type=jax.ShapeDtypeStruct((BS, D), jnp.float32),
        scratch_types=[
            pltpu.VMEM((BC, D), jnp.float32),
            pltpu.VMEM((BC, D), jnp.float32),
            pltpu.VMEM((_C, D), jnp.float32),
            pltpu.VMEM((_C, D), jnp.float32),
            pltpu.SemaphoreType.DMA,
            pltpu.SemaphoreType.DMA,
            pltpu.SemaphoreType.DMA,
            pltpu.SemaphoreType.DMA,
        ],
        compiler_params=pltpu.CompilerParams(use_tc_tiling_on_sc=True),
    )
    def sc_add(x_hbm, t_hbm, out_hbm, xb0, xb1, tb0, tb1, is0, is1, os0, os1):
        wid = lax.axis_index("s") * _NC + lax.axis_index("c")
        s0 = wid * s_per_w
        xbufs, tbufs = (xb0, xb1), (tb0, tb1)
        isems, osems = (is0, is1), (os0, os1)

        def start_in(j, b):
            sbase = s0 + j * _C
            pltpu.async_copy(t_hbm.at[pl.ds(sbase, _C)], tbufs[b], isems[b])
            for bb in range(B):
                pltpu.async_copy(
                    x_hbm.at[pl.ds(bb * S + sbase, _C)],
                    xbufs[b].at[pl.ds(bb * _C, _C)], isems[b])

        def wait_in(j, b):
            sbase = s0 + j * _C
            pltpu.make_async_copy(
                t_hbm.at[pl.ds(sbase, _C)], tbufs[b], isems[b]).wait()
            for bb in range(B):
                pltpu.make_async_copy(
                    x_hbm.at[pl.ds(bb * S + sbase, _C)],
                    xbufs[b].at[pl.ds(bb * _C, _C)], isems[b]).wait()

        def start_out(j, b):
            sbase = s0 + j * _C
            for bb in range(B):
                pltpu.async_copy(
                    xbufs[b].at[pl.ds(bb * _C, _C)],
                    out_hbm.at[pl.ds(bb * S + sbase, _C)], osems[b])

        def wait_out(j, b):
            sbase = s0 + j * _C
            for bb in range(B):
                pltpu.make_async_copy(
                    xbufs[b].at[pl.ds(bb * _C, _C)],
                    out_hbm.at[pl.ds(bb * S + sbase, _C)], osems[b]).wait()

        start_in(0, 0)

        def half_step(jj, b):
            j = jj * 2 + b
            xb, tb = xbufs[b], tbufs[b]

            @pl.when(j + 1 < chunks)
            def _():
                @pl.when(j >= 1)
                def _():
                    wait_out(j - 1, 1 - b)
                start_in(j + 1, 1 - b)

            wait_in(j, b)

            @plsc.parallel_loop(0, D, 16, unroll=4)
            def _(i):
                sl = pl.ds(i, 16)
                for r in range(BC):
                    xb[r, sl] = xb[r, sl] + tb[r % _C, sl]

            start_out(j, b)

        def body(jj, carry):
            half_step(jj, 0)
            half_step(jj, 1)
            return carry

        lax.fori_loop(0, chunks // 2, body, 0)
        wait_out(chunks - 2, 0)
        wait_out(chunks - 1, 1)

    out = sc_add(input_embeddings.reshape(BS, D), table)
    return out.reshape(B, S, D)
